# baseline (speedup 1.0000x reference)
import functools

import jax
import jax.numpy as jnp
from jax import lax
from jax.experimental import pallas as pl
from jax.experimental.pallas import tpu as pltpu
from jax.experimental.pallas import tpu_sc as plsc

POOL_SIZE = 8
VOCAB = 1024
N_CB = 4
SEG_LEN = 96
NUM_CODES = 1024
EMB_DIM = 128
LATENT = 256
HIDDEN = 512
BETA = 0.1
B = 64

BB = 32

_SC_CH = 768


def _build_embed_gather(total_rows):
    info = plsc.get_sparse_core_info()
    nw = info.num_cores * info.num_subcores
    per_w = total_rows // nw
    nch = per_w // _SC_CH
    mesh = plsc.VectorSubcoreMesh(core_axis_name="c", subcore_axis_name="s")

    @functools.partial(
        pl.kernel, mesh=mesh,
        out_type=jax.ShapeDtypeStruct((total_rows, EMB_DIM), jnp.float32),
        scratch_types=[
            pltpu.VMEM((per_w,), jnp.int32),
            pltpu.VMEM((_SC_CH, EMB_DIM), jnp.float32),
            pltpu.SemaphoreType.DMA,
        ],
    )
    def gather(emb_hbm, idx_hbm, out_hbm, idx_v, r0, s0):
        wid = lax.axis_index("s") * info.num_cores + lax.axis_index("c")
        base = wid * per_w
        pltpu.sync_copy(idx_hbm.at[pl.ds(base, per_w)], idx_v)
        for i in range(nch):
            pltpu.async_copy(
                emb_hbm.at[idx_v.at[pl.ds(i * _SC_CH, _SC_CH)]], r0,
                s0).wait()
            pltpu.sync_copy(r0, out_hbm.at[pl.ds(base + i * _SC_CH, _SC_CH)])

    return gather


def _encoder_vq_body(x_ref, c1m_ref, c1b_ref, c2m_ref, c2b_ref, c3t_ref,
                     c3b_ref, cbt_ref, cb_ref, ze_ref, zq_ref):
    f32 = jnp.float32
    x = x_ref[...]

    def conv(xin, wm_ref, b_ref):
        z = jnp.dot(xin.reshape(BB * SEG_LEN, HIDDEN).astype(jnp.bfloat16),
                    wm_ref[...].astype(jnp.bfloat16),
                    preferred_element_type=f32)
        z = z.reshape(BB, SEG_LEN, 3 * HIDDEN)
        z0 = z[:, :, :HIDDEN]
        z1 = z[:, :, HIDDEN:2 * HIDDEN]
        z2 = z[:, :, 2 * HIDDEN:]
        zrow = jnp.zeros((BB, 1, HIDDEN), f32)
        y = (z1
             + jnp.concatenate([zrow, z0[:, :-1, :]], axis=1)
             + jnp.concatenate([z2[:, 1:, :], zrow], axis=1)
             + b_ref[...].reshape(1, 1, HIDDEN))
        return jnp.maximum(y, 0.0)

    y = conv(x, c1m_ref, c1b_ref)
    y = conv(y, c2m_ref, c2b_ref)
    p = y.reshape(BB, POOL_SIZE, SEG_LEN // POOL_SIZE, HIDDEN).mean(axis=2)
    ze = (jnp.dot(p.reshape(BB * POOL_SIZE, HIDDEN), c3t_ref[...],
                  preferred_element_type=f32)
          + c3b_ref[...].reshape(1, LATENT))

    cbt = cbt_ref[...]
    cbsq = jnp.sum(cbt * cbt, axis=0).reshape(1, NUM_CODES)
    cross = jnp.dot(ze, cbt, preferred_element_type=f32)
    dist = jnp.sum(ze * ze, axis=1, keepdims=True) - 2.0 * cross + cbsq
    md = jnp.min(dist, axis=1, keepdims=True)
    iota = jax.lax.broadcasted_iota(jnp.int32, (BB * POOL_SIZE, NUM_CODES), 1)
    code = jnp.min(jnp.where(dist <= md, iota, NUM_CODES), axis=1,
                   keepdims=True)
    onehot = (iota == code).astype(f32)
    zq = jnp.dot(onehot, cb_ref[...], preferred_element_type=f32)

    ze_ref[...] = ze.reshape(BB, POOL_SIZE, LATENT)
    zq_ref[...] = zq.reshape(BB, POOL_SIZE, LATENT)


def _build_label_gather():
    info = plsc.get_sparse_core_info()
    nw = info.num_cores * info.num_subcores
    mesh = plsc.VectorSubcoreMesh(core_axis_name="c", subcore_axis_name="s")
    per_w = (B * SEG_LEN * N_CB) // nw

    @functools.partial(
        pl.kernel, mesh=mesh,
        out_type=jax.ShapeDtypeStruct((nw, 16), jnp.float32),
        scratch_types=[
            pltpu.VMEM((per_w,), jnp.int32),
            pltpu.VMEM((per_w,), jnp.float32),
            pltpu.VMEM((16,), jnp.float32),
            pltpu.SemaphoreType.DMA,
        ],
    )
    def gather(ltab_hbm, idx_hbm, out_hbm, idx_v, vals_v, part_v, sem):
        wid = lax.axis_index("s") * info.num_cores + lax.axis_index("c")
        base = wid * per_w
        pltpu.sync_copy(idx_hbm.at[pl.ds(base, per_w)], idx_v)
        pltpu.async_copy(ltab_hbm.at[idx_v], vals_v, sem).wait()
        acc = jnp.zeros((16,), jnp.float32)
        for i in range(per_w // 16):
            acc = acc + vals_v[pl.ds(i * 16, 16)]
        part_v[...] = acc
        pltpu.sync_copy(part_v, out_hbm.at[wid])

    return gather


def _decoder_loss_body(zep_ref, zqp_ref, zec_ref, zqc_ref, zen_ref, zqn_ref,
                       lab_ref, fc1t_ref, fc1b_ref, fc2t_ref,
                       fc2b_ref, d1s_ref, d1b_ref, d2m_ref, d2b_ref,
                       logits_ref, idx_ref, out_ref):
    f32 = jnp.float32
    commit = (jnp.sum((zep_ref[...] - zqp_ref[...]) ** 2)
              + jnp.sum((zec_ref[...] - zqc_ref[...]) ** 2)
              + jnp.sum((zen_ref[...] - zqn_ref[...]) ** 2))

    hp = zqp_ref[...].reshape(B, POOL_SIZE * LATENT)
    hc = zqc_ref[...].reshape(B, POOL_SIZE * LATENT)
    hn = zqn_ref[...].reshape(B, POOL_SIZE * LATENT)
    h0 = jnp.concatenate([hp, hc, hn], axis=1)

    dn_t = (((1,), (1,)), ((), ()))
    h1 = jnp.maximum(lax.dot_general(h0, fc1t_ref[...], dn_t,
                                     preferred_element_type=f32)
                     + fc1b_ref[...].reshape(1, HIDDEN), 0.0)
    h2 = jnp.maximum(lax.dot_general(h1, fc2t_ref[...], dn_t,
                                     preferred_element_type=f32)
                     + fc2b_ref[...].reshape(1, HIDDEN), 0.0)

    d1 = d1s_ref[...]
    a0 = d1[0] + d1[1]
    a1 = d1[0] + d1[1] + d1[2]
    a2 = d1[1] + d1[2]
    d1b = d1b_ref[...].reshape(1, HIDDEN)
    x0 = jnp.maximum(jnp.dot(h2, a0, preferred_element_type=f32) + d1b, 0.0)
    x1 = jnp.maximum(jnp.dot(h2, a1, preferred_element_type=f32) + d1b, 0.0)
    x2 = jnp.maximum(jnp.dot(h2, a2, preferred_element_type=f32) + d1b, 0.0)
    xcat = jnp.concatenate([x0, x1, x2], axis=0)
    logits = (jnp.dot(xcat, d2m_ref[...], preferred_element_type=f32)
              + d2b_ref[...].reshape(1, N_CB * VOCAB))

    logits_ref[...] = logits

    lab = lab_ref[...]
    t_i = jax.lax.broadcasted_iota(jnp.int32, (B, SEG_LEN, N_CB), 1)
    b_i = jax.lax.broadcasted_iota(jnp.int32, (B, SEG_LEN, N_CB), 0)
    c_i = jax.lax.broadcasted_iota(jnp.int32, (B, SEG_LEN, N_CB), 2)
    cls = jnp.where(t_i == 0, 0, jnp.where(t_i == SEG_LEN - 1, 2, 1))
    idx_ref[...] = (cls * B + b_i) * (N_CB * VOCAB) + c_i * VOCAB + lab

    row = jax.lax.broadcasted_iota(jnp.int32, (3 * B, 1), 0)
    wrow = jnp.where((row >= B) & (row < 2 * B), 94.0, 1.0)

    total_logz = jnp.zeros((), f32)
    for c in range(N_CB):
        lc = logits[:, c * VOCAB:(c + 1) * VOCAB]
        m = jnp.max(lc, axis=1, keepdims=True)
        s = jnp.sum(jnp.exp(lc - m), axis=1, keepdims=True)
        logz = m + jnp.log(s)
        total_logz += jnp.sum(wrow * logz)

    partial = (total_logz / (B * SEG_LEN * N_CB)
               + BETA * commit / (B * POOL_SIZE * LATENT))
    out_ref[...] = jnp.reshape(partial, (1, 1))


@jax.jit
def _run(tokens_prev, tokens_curr, tokens_next, emb, c1w, c1b, c2w, c2b, c3w,
         c3b, codebook, fc1w, fc1b, fc2w, fc2b, d1w, d1b, d2w, d2b):
    f32 = jnp.float32
    rows_per_set = B * SEG_LEN * N_CB
    egather = _build_embed_gather(rows_per_set)
    xs = [egather(emb, t.reshape(-1)).reshape(B, SEG_LEN, N_CB * EMB_DIM)
          for t in (tokens_prev, tokens_curr, tokens_next)]

    c1m = jnp.transpose(c1w, (1, 2, 0)).reshape(HIDDEN, 3 * HIDDEN)
    c1m = c1m.astype(jnp.bfloat16)
    c2m = jnp.transpose(c2w, (1, 2, 0)).reshape(HIDDEN, 3 * HIDDEN)
    c2m = c2m.astype(jnp.bfloat16)
    c3t = c3w.T
    cbt = codebook.T

    enc = pl.pallas_call(
        _encoder_vq_body,
        grid=(B // BB,),
        in_specs=[
            pl.BlockSpec((BB, SEG_LEN, N_CB * EMB_DIM), lambda i: (i, 0, 0)),
            pl.BlockSpec((HIDDEN, 3 * HIDDEN), lambda i: (0, 0)),
            pl.BlockSpec((1, HIDDEN), lambda i: (0, 0)),
            pl.BlockSpec((HIDDEN, 3 * HIDDEN), lambda i: (0, 0)),
            pl.BlockSpec((1, HIDDEN), lambda i: (0, 0)),
            pl.BlockSpec((HIDDEN, LATENT), lambda i: (0, 0)),
            pl.BlockSpec((1, LATENT), lambda i: (0, 0)),
            pl.BlockSpec((LATENT, NUM_CODES), lambda i: (0, 0)),
            pl.BlockSpec((NUM_CODES, LATENT), lambda i: (0, 0)),
        ],
        out_specs=[
            pl.BlockSpec((BB, POOL_SIZE, LATENT), lambda i: (i, 0, 0)),
            pl.BlockSpec((BB, POOL_SIZE, LATENT), lambda i: (i, 0, 0)),
        ],
        out_shape=[
            jax.ShapeDtypeStruct((B, POOL_SIZE, LATENT), f32),
            jax.ShapeDtypeStruct((B, POOL_SIZE, LATENT), f32),
        ],
    )
    pairs = [enc(xset, c1m, c1b.reshape(1, HIDDEN), c2m,
                 c2b.reshape(1, HIDDEN), c3t, c3b.reshape(1, LATENT), cbt,
                 codebook) for xset in xs]

    d1s = jnp.transpose(d1w, (2, 0, 1))
    logits, idx, part = pl.pallas_call(
        _decoder_loss_body,
        out_shape=[
            jax.ShapeDtypeStruct((3 * B, N_CB * VOCAB), f32),
            jax.ShapeDtypeStruct((B, SEG_LEN, N_CB), jnp.int32),
            jax.ShapeDtypeStruct((1, 1), f32),
        ],
    )(pairs[0][0], pairs[0][1], pairs[1][0], pairs[1][1], pairs[2][0],
      pairs[2][1], tokens_curr, fc1w, fc1b.reshape(1, HIDDEN), fc2w,
      fc2b.reshape(1, HIDDEN), d1s, d1b.reshape(1, HIDDEN), d2w[:, :, 0],
      d2b.reshape(1, N_CB * VOCAB))

    parts = _build_label_gather()(logits.reshape(-1), idx.reshape(-1))
    return part[0, 0] - jnp.sum(parts) / (B * SEG_LEN * N_CB)


def kernel(tokens_prev, tokens_curr, tokens_next, emb, c1w, c1b, c2w, c2b,
           c3w, c3b, codebook, fc1w, fc1b, fc2w, fc2b, d1w, d1b, d2w, d2b):
    return _run(tokens_prev, tokens_curr, tokens_next, emb, c1w, c1b, c2w,
                c2b, c3w, c3b, codebook, fc1w, fc1b, fc2w, fc2b, d1w, d1b,
                d2w, d2b)


# speedup vs baseline: 1.0439x; 1.0439x over previous
import functools

import jax
import jax.numpy as jnp
from jax import lax
from jax.experimental import pallas as pl
from jax.experimental.pallas import tpu as pltpu
from jax.experimental.pallas import tpu_sc as plsc

POOL_SIZE = 8
VOCAB = 1024
N_CB = 4
SEG_LEN = 96
NUM_CODES = 1024
EMB_DIM = 128
LATENT = 256
HIDDEN = 512
BETA = 0.1
B = 64

BB = 16

_SC_CH = 768


def _build_embed_gather(total_rows):
    info = plsc.get_sparse_core_info()
    nw = info.num_cores * info.num_subcores
    per_w = total_rows // nw
    nch = per_w // _SC_CH
    mesh = plsc.VectorSubcoreMesh(core_axis_name="c", subcore_axis_name="s")

    @functools.partial(
        pl.kernel, mesh=mesh,
        out_type=jax.ShapeDtypeStruct((total_rows, EMB_DIM), jnp.float32),
        scratch_types=[
            pltpu.VMEM((per_w,), jnp.int32),
            pltpu.VMEM((_SC_CH, EMB_DIM), jnp.float32),
            pltpu.SemaphoreType.DMA,
        ],
    )
    def gather(emb_hbm, idx_hbm, out_hbm, idx_v, r0, s0):
        wid = lax.axis_index("s") * info.num_cores + lax.axis_index("c")
        base = wid * per_w
        pltpu.sync_copy(idx_hbm.at[pl.ds(base, per_w)], idx_v)
        for i in range(nch):
            pltpu.async_copy(
                emb_hbm.at[idx_v.at[pl.ds(i * _SC_CH, _SC_CH)]], r0,
                s0).wait()
            pltpu.sync_copy(r0, out_hbm.at[pl.ds(base + i * _SC_CH, _SC_CH)])

    return gather


def _encoder_vq_body(x_ref, c1m_ref, c1b_ref, c2m_ref, c2b_ref, c3t_ref,
                     c3b_ref, cbt_ref, cb_ref, ze_ref, zq_ref):
    f32 = jnp.float32
    x = x_ref[...]

    def conv(xin, wm_ref, b_ref):
        z = jnp.dot(xin.reshape(BB * SEG_LEN, HIDDEN).astype(jnp.bfloat16),
                    wm_ref[...].astype(jnp.bfloat16),
                    preferred_element_type=f32)
        z = z.reshape(BB, SEG_LEN, 3 * HIDDEN)
        z0 = z[:, :, :HIDDEN]
        z1 = z[:, :, HIDDEN:2 * HIDDEN]
        z2 = z[:, :, 2 * HIDDEN:]
        zrow = jnp.zeros((BB, 1, HIDDEN), f32)
        y = (z1
             + jnp.concatenate([zrow, z0[:, :-1, :]], axis=1)
             + jnp.concatenate([z2[:, 1:, :], zrow], axis=1)
             + b_ref[...].reshape(1, 1, HIDDEN))
        return jnp.maximum(y, 0.0)

    y = conv(x, c1m_ref, c1b_ref)
    y = conv(y, c2m_ref, c2b_ref)
    p = y.reshape(BB, POOL_SIZE, SEG_LEN // POOL_SIZE, HIDDEN).mean(axis=2)
    ze = (jnp.dot(p.reshape(BB * POOL_SIZE, HIDDEN), c3t_ref[...],
                  preferred_element_type=f32)
          + c3b_ref[...].reshape(1, LATENT))

    cbt = cbt_ref[...]
    cbsq = jnp.sum(cbt * cbt, axis=0).reshape(1, NUM_CODES)
    cross = jnp.dot(ze, cbt, preferred_element_type=f32)
    dist = jnp.sum(ze * ze, axis=1, keepdims=True) - 2.0 * cross + cbsq
    md = jnp.min(dist, axis=1, keepdims=True)
    iota = jax.lax.broadcasted_iota(jnp.int32, (BB * POOL_SIZE, NUM_CODES), 1)
    code = jnp.min(jnp.where(dist <= md, iota, NUM_CODES), axis=1,
                   keepdims=True)
    onehot = (iota == code).astype(f32)
    zq = jnp.dot(onehot, cb_ref[...], preferred_element_type=f32)

    ze_ref[...] = ze.reshape(BB, POOL_SIZE, LATENT)
    zq_ref[...] = zq.reshape(BB, POOL_SIZE, LATENT)


def _build_label_gather():
    info = plsc.get_sparse_core_info()
    nw = info.num_cores * info.num_subcores
    mesh = plsc.VectorSubcoreMesh(core_axis_name="c", subcore_axis_name="s")
    per_w = (B * SEG_LEN * N_CB) // nw

    @functools.partial(
        pl.kernel, mesh=mesh,
        out_type=jax.ShapeDtypeStruct((nw, 16), jnp.float32),
        scratch_types=[
            pltpu.VMEM((per_w,), jnp.int32),
            pltpu.VMEM((per_w,), jnp.float32),
            pltpu.VMEM((16,), jnp.float32),
            pltpu.SemaphoreType.DMA,
        ],
    )
    def gather(ltab_hbm, idx_hbm, out_hbm, idx_v, vals_v, part_v, sem):
        wid = lax.axis_index("s") * info.num_cores + lax.axis_index("c")
        base = wid * per_w
        pltpu.sync_copy(idx_hbm.at[pl.ds(base, per_w)], idx_v)
        pltpu.async_copy(ltab_hbm.at[idx_v], vals_v, sem).wait()
        acc = jnp.zeros((16,), jnp.float32)
        for i in range(per_w // 16):
            acc = acc + vals_v[pl.ds(i * 16, 16)]
        part_v[...] = acc
        pltpu.sync_copy(part_v, out_hbm.at[wid])

    return gather


def _decoder_loss_body(zep_ref, zqp_ref, zec_ref, zqc_ref, zen_ref, zqn_ref,
                       lab_ref, fc1t_ref, fc1b_ref, fc2t_ref,
                       fc2b_ref, d1s_ref, d1b_ref, d2m_ref, d2b_ref,
                       logits_ref, idx_ref, out_ref):
    f32 = jnp.float32
    commit = (jnp.sum((zep_ref[...] - zqp_ref[...]) ** 2)
              + jnp.sum((zec_ref[...] - zqc_ref[...]) ** 2)
              + jnp.sum((zen_ref[...] - zqn_ref[...]) ** 2))

    hp = zqp_ref[...].reshape(B, POOL_SIZE * LATENT)
    hc = zqc_ref[...].reshape(B, POOL_SIZE * LATENT)
    hn = zqn_ref[...].reshape(B, POOL_SIZE * LATENT)
    h0 = jnp.concatenate([hp, hc, hn], axis=1)

    dn_t = (((1,), (1,)), ((), ()))
    h1 = jnp.maximum(lax.dot_general(h0, fc1t_ref[...], dn_t,
                                     preferred_element_type=f32)
                     + fc1b_ref[...].reshape(1, HIDDEN), 0.0)
    h2 = jnp.maximum(lax.dot_general(h1, fc2t_ref[...], dn_t,
                                     preferred_element_type=f32)
                     + fc2b_ref[...].reshape(1, HIDDEN), 0.0)

    d1 = d1s_ref[...]
    a0 = d1[0] + d1[1]
    a1 = d1[0] + d1[1] + d1[2]
    a2 = d1[1] + d1[2]
    d1b = d1b_ref[...].reshape(1, HIDDEN)
    x0 = jnp.maximum(jnp.dot(h2, a0, preferred_element_type=f32) + d1b, 0.0)
    x1 = jnp.maximum(jnp.dot(h2, a1, preferred_element_type=f32) + d1b, 0.0)
    x2 = jnp.maximum(jnp.dot(h2, a2, preferred_element_type=f32) + d1b, 0.0)
    xcat = jnp.concatenate([x0, x1, x2], axis=0)
    logits = (jnp.dot(xcat, d2m_ref[...], preferred_element_type=f32)
              + d2b_ref[...].reshape(1, N_CB * VOCAB))

    logits_ref[...] = logits

    lab = lab_ref[...]
    t_i = jax.lax.broadcasted_iota(jnp.int32, (B, SEG_LEN, N_CB), 1)
    b_i = jax.lax.broadcasted_iota(jnp.int32, (B, SEG_LEN, N_CB), 0)
    c_i = jax.lax.broadcasted_iota(jnp.int32, (B, SEG_LEN, N_CB), 2)
    cls = jnp.where(t_i == 0, 0, jnp.where(t_i == SEG_LEN - 1, 2, 1))
    idx_ref[...] = (cls * B + b_i) * (N_CB * VOCAB) + c_i * VOCAB + lab

    row = jax.lax.broadcasted_iota(jnp.int32, (3 * B, 1), 0)
    wrow = jnp.where((row >= B) & (row < 2 * B), 94.0, 1.0)

    total_logz = jnp.zeros((), f32)
    for c in range(N_CB):
        lc = logits[:, c * VOCAB:(c + 1) * VOCAB]
        m = jnp.max(lc, axis=1, keepdims=True)
        s = jnp.sum(jnp.exp(lc - m), axis=1, keepdims=True)
        logz = m + jnp.log(s)
        total_logz += jnp.sum(wrow * logz)

    partial = (total_logz / (B * SEG_LEN * N_CB)
               + BETA * commit / (B * POOL_SIZE * LATENT))
    out_ref[...] = jnp.reshape(partial, (1, 1))


@jax.jit
def _run(tokens_prev, tokens_curr, tokens_next, emb, c1w, c1b, c2w, c2b, c3w,
         c3b, codebook, fc1w, fc1b, fc2w, fc2b, d1w, d1b, d2w, d2b):
    f32 = jnp.float32
    rows_per_set = B * SEG_LEN * N_CB
    egather = _build_embed_gather(rows_per_set)
    xs = [egather(emb, t.reshape(-1)).reshape(B, SEG_LEN, N_CB * EMB_DIM)
          for t in (tokens_prev, tokens_curr, tokens_next)]

    c1m = jnp.transpose(c1w, (1, 2, 0)).reshape(HIDDEN, 3 * HIDDEN)
    c1m = c1m.astype(jnp.bfloat16)
    c2m = jnp.transpose(c2w, (1, 2, 0)).reshape(HIDDEN, 3 * HIDDEN)
    c2m = c2m.astype(jnp.bfloat16)
    c3t = c3w.T
    cbt = codebook.T

    enc = pl.pallas_call(
        _encoder_vq_body,
        grid=(B // BB,),
        in_specs=[
            pl.BlockSpec((BB, SEG_LEN, N_CB * EMB_DIM), lambda i: (i, 0, 0)),
            pl.BlockSpec((HIDDEN, 3 * HIDDEN), lambda i: (0, 0)),
            pl.BlockSpec((1, HIDDEN), lambda i: (0, 0)),
            pl.BlockSpec((HIDDEN, 3 * HIDDEN), lambda i: (0, 0)),
            pl.BlockSpec((1, HIDDEN), lambda i: (0, 0)),
            pl.BlockSpec((HIDDEN, LATENT), lambda i: (0, 0)),
            pl.BlockSpec((1, LATENT), lambda i: (0, 0)),
            pl.BlockSpec((LATENT, NUM_CODES), lambda i: (0, 0)),
            pl.BlockSpec((NUM_CODES, LATENT), lambda i: (0, 0)),
        ],
        out_specs=[
            pl.BlockSpec((BB, POOL_SIZE, LATENT), lambda i: (i, 0, 0)),
            pl.BlockSpec((BB, POOL_SIZE, LATENT), lambda i: (i, 0, 0)),
        ],
        out_shape=[
            jax.ShapeDtypeStruct((B, POOL_SIZE, LATENT), f32),
            jax.ShapeDtypeStruct((B, POOL_SIZE, LATENT), f32),
        ],
    )
    pairs = [enc(xset, c1m, c1b.reshape(1, HIDDEN), c2m,
                 c2b.reshape(1, HIDDEN), c3t, c3b.reshape(1, LATENT), cbt,
                 codebook) for xset in xs]

    d1s = jnp.transpose(d1w, (2, 0, 1))
    logits, idx, part = pl.pallas_call(
        _decoder_loss_body,
        out_shape=[
            jax.ShapeDtypeStruct((3 * B, N_CB * VOCAB), f32),
            jax.ShapeDtypeStruct((B, SEG_LEN, N_CB), jnp.int32),
            jax.ShapeDtypeStruct((1, 1), f32),
        ],
    )(pairs[0][0], pairs[0][1], pairs[1][0], pairs[1][1], pairs[2][0],
      pairs[2][1], tokens_curr, fc1w, fc1b.reshape(1, HIDDEN), fc2w,
      fc2b.reshape(1, HIDDEN), d1s, d1b.reshape(1, HIDDEN), d2w[:, :, 0],
      d2b.reshape(1, N_CB * VOCAB))

    parts = _build_label_gather()(logits.reshape(-1), idx.reshape(-1))
    return part[0, 0] - jnp.sum(parts) / (B * SEG_LEN * N_CB)


def kernel(tokens_prev, tokens_curr, tokens_next, emb, c1w, c1b, c2w, c2b,
           c3w, c3b, codebook, fc1w, fc1b, fc2w, fc2b, d1w, d1b, d2w, d2b):
    return _run(tokens_prev, tokens_curr, tokens_next, emb, c1w, c1b, c2w,
                c2b, c3w, c3b, codebook, fc1w, fc1b, fc2w, fc2b, d1w, d1b,
                d2w, d2b)
